# baseline (speedup 1.0000x reference)
import jax
import jax.numpy as jnp
from jax import lax
from jax.experimental import pallas as pl
from jax.experimental.pallas import tpu as pltpu
from jax.experimental.pallas import tpu_sc as plsc

IN_DIM = 3
EMB = 32
STEPS_PER_DAY = 288
DOW = 7
NC = 2
NS = 16
NW = NC * NS
L = 16

B, S, V = 64, 12, 1024
BT, BR = 8, 8
VT, VC = 8, 128
DT, DR = 12, 8
SLABS = S * BT
SLABS_PER_W = SLABS // NW


def _tec_body(x6, wb_hbm, bb_hbm, todf_hbm, dowf_hbm, out6,
              xb0, xb1, xb2, tmap, dmap, blk0, blk1,
              wv, bv, todv, dowv, semx, semt, sem0, sem1):
    wid = lax.axis_index("s") * NC + lax.axis_index("c")

    cp_tab = [
        pltpu.async_copy(wb_hbm, wv, semt),
        pltpu.async_copy(bb_hbm, bv, semt),
        pltpu.async_copy(todf_hbm, todv, semt),
        pltpu.async_copy(dowf_hbm, dowv, semt),
    ]
    for cp in cp_tab:
        cp.wait()

    iota = lax.iota(jnp.int32, L)
    blks = [blk0, blk1]
    sems = [sem0, sem1]

    def slab_body(k, carry):
        slab = wid * SLABS_PER_W + k
        s = slab // BT
        bt = slab % BT

        cpx = [
            pltpu.async_copy(x6.at[s, 0, bt], xb0, semx),
            pltpu.async_copy(x6.at[s, 1, bt], xb1, semx),
            pltpu.async_copy(x6.at[s, 2, bt], xb2, semx),
        ]
        for cp in cpx:
            cp.wait()

        def br_body(br, carry):
            b = bt * BR + br

            @plsc.parallel_loop(0, VT * 8, unroll=4)
            def idx_body(g2):
                vt = g2 // 8
                g = g2 % 8
                x1v = xb1[vt, br, pl.ds(g * L, L)]
                x2v = xb2[vt, br, pl.ds(g * L, L)]
                tmap[pl.ds(g2 * L, L)] = (x1v * float(STEPS_PER_DAY)).astype(jnp.int32)
                dmap[pl.ds(g2 * L, L)] = x2v.astype(jnp.int32)

            for dt in range(DT):
                blk = blks[dt % 2]
                if dt >= 2:
                    pltpu.make_async_copy(blk, out6.at[b, s, dt - 2],
                                          sems[dt % 2]).wait()
                if dt < 4:
                    wrows = [[wv[dt * DR + dr, j] for j in range(IN_DIM)]
                             for dr in range(DR)]
                    brows = [bv[dt * DR + dr] for dr in range(DR)]

                    @plsc.parallel_loop(0, VT * 8, unroll=4)
                    def feat_body(g2):
                        vt = g2 // 8
                        g = g2 % 8
                        x0v = xb0[vt, br, pl.ds(g * L, L)]
                        x1v = xb1[vt, br, pl.ds(g * L, L)]
                        x2v = xb2[vt, br, pl.ds(g * L, L)]
                        fs = [w[0] * x0v + w[1] * x1v + w[2] * x2v + bd
                              for w, bd in zip(wrows, brows)]
                        for dr in range(DR):
                            blk[vt, dr, pl.ds(g * L, L)] = fs[dr]
                else:
                    tab = todv if dt < 8 else dowv
                    imap = tmap if dt < 8 else dmap
                    rows = STEPS_PER_DAY if dt < 8 else DOW
                    dbase = (dt - 4) * DR if dt < 8 else (dt - 8) * DR

                    @plsc.parallel_loop(0, VT * 8, unroll=4)
                    def gat_body(g2):
                        vt = g2 // 8
                        g = g2 % 8
                        iv = imap[pl.ds(g2 * L, L)]
                        rs = [plsc.load_gather(tab, [iv + (dbase + dr) * rows])
                              for dr in range(DR)]
                        for dr in range(DR):
                            blk[vt, dr, pl.ds(g * L, L)] = rs[dr]

                pltpu.async_copy(blk, out6.at[b, s, dt], sems[dt % 2])

            pltpu.make_async_copy(blks[0], out6.at[b, s, DT - 2], sems[0]).wait()
            pltpu.make_async_copy(blks[1], out6.at[b, s, DT - 1], sems[1]).wait()
            return carry
        lax.fori_loop(0, BR, br_body, 0)
        return carry

    lax.fori_loop(0, SLABS_PER_W, slab_body, 0)


def kernel(x, W, b, tod_table, dow_table):
    x6 = (x.transpose(1, 3, 0, 2)
           .reshape(S, IN_DIM, BT, BR, VT, VC)
           .transpose(0, 1, 2, 4, 3, 5))
    wb = jnp.broadcast_to(W[:, :, None], (EMB, IN_DIM, L))
    bb = jnp.broadcast_to(b[:, None], (EMB, L))
    todf = tod_table.T.reshape(STEPS_PER_DAY * EMB)
    dowf = dow_table.T.reshape(DOW * EMB)

    mesh = plsc.VectorSubcoreMesh(core_axis_name="c", subcore_axis_name="s")
    run = pl.kernel(
        _tec_body,
        out_type=jax.ShapeDtypeStruct((B, S, DT, VT, DR, VC), jnp.float32),
        mesh=mesh,
        compiler_params=pltpu.CompilerParams(needs_layout_passes=False,
                                             use_tc_tiling_on_sc=False),
        scratch_types=[
            pltpu.VMEM((VT, BR, VC), jnp.float32),
            pltpu.VMEM((VT, BR, VC), jnp.float32),
            pltpu.VMEM((VT, BR, VC), jnp.float32),
            pltpu.VMEM((V,), jnp.int32),
            pltpu.VMEM((V,), jnp.int32),
            pltpu.VMEM((VT, DR, VC), jnp.float32),
            pltpu.VMEM((VT, DR, VC), jnp.float32),
            pltpu.VMEM((EMB, IN_DIM, L), jnp.float32),
            pltpu.VMEM((EMB, L), jnp.float32),
            pltpu.VMEM((STEPS_PER_DAY * EMB,), jnp.float32),
            pltpu.VMEM((DOW * EMB,), jnp.float32),
            pltpu.SemaphoreType.DMA,
            pltpu.SemaphoreType.DMA,
            pltpu.SemaphoreType.DMA,
            pltpu.SemaphoreType.DMA,
        ],
    )
    out6 = run(x6, wb, bb, todf, dowf)
    return (out6.transpose(0, 1, 3, 5, 2, 4)
                .reshape(B, S, V, IN_DIM * EMB))


# speedup vs baseline: 1.1972x; 1.1972x over previous
import jax
import jax.numpy as jnp
from jax import lax
from jax.experimental import pallas as pl
from jax.experimental.pallas import tpu as pltpu
from jax.experimental.pallas import tpu_sc as plsc

IN_DIM = 3
EMB = 32
STEPS_PER_DAY = 288
DOW = 7
NC = 2
NS = 16
NW = NC * NS
L = 16

B, S, V = 64, 12, 1024
BT, BR = 8, 8
VT, VC = 8, 128
DT, DR = 12, 8
SLABS = S * BT
SLABS_PER_W = SLABS // NW


def _tec_body(x6, wb_hbm, bb_hbm, todf_hbm, dowf_hbm, out6,
              xb0, xb1, xb2, tmap, dmap, blk0, blk1, blk2,
              wv, bv, todv, dowv, semx, semt, sem0, sem1, sem2):
    wid = lax.axis_index("s") * NC + lax.axis_index("c")

    cp_tab = [
        pltpu.async_copy(wb_hbm, wv, semt),
        pltpu.async_copy(bb_hbm, bv, semt),
        pltpu.async_copy(todf_hbm, todv, semt),
        pltpu.async_copy(dowf_hbm, dowv, semt),
    ]
    for cp in cp_tab:
        cp.wait()

    iota = lax.iota(jnp.int32, L)
    blks = [blk0, blk1, blk2]
    sems = [sem0, sem1, sem2]

    def slab_body(k, carry):
        slab = wid * SLABS_PER_W + k
        s = slab // BT
        bt = slab % BT

        cpx = [
            pltpu.async_copy(x6.at[s, 0, bt], xb0, semx),
            pltpu.async_copy(x6.at[s, 1, bt], xb1, semx),
            pltpu.async_copy(x6.at[s, 2, bt], xb2, semx),
        ]
        for cp in cpx:
            cp.wait()

        def br_body(br, carry):
            b = bt * BR + br

            @plsc.parallel_loop(0, VT * 8, unroll=2)
            def idx_body(g2):
                vt = g2 // 8
                g = g2 % 8
                x1v = xb1[vt, br, pl.ds(g * L, L)]
                x2v = xb2[vt, br, pl.ds(g * L, L)]
                tmap[pl.ds(g2 * L, L)] = (x1v * float(STEPS_PER_DAY)).astype(jnp.int32)
                dmap[pl.ds(g2 * L, L)] = x2v.astype(jnp.int32)

            for dt in range(DT):
                blk = blks[dt % 3]
                if dt >= 3:
                    pltpu.make_async_copy(blk, out6.at[b, s, dt - 3],
                                          sems[dt % 3]).wait()
                if dt < 4:
                    wrows = [[wv[dt * DR + dr, j] for j in range(IN_DIM)]
                             for dr in range(DR)]
                    brows = [bv[dt * DR + dr] for dr in range(DR)]

                    @plsc.parallel_loop(0, VT * 8, unroll=2)
                    def feat_body(g2):
                        vt = g2 // 8
                        g = g2 % 8
                        x0v = xb0[vt, br, pl.ds(g * L, L)]
                        x1v = xb1[vt, br, pl.ds(g * L, L)]
                        x2v = xb2[vt, br, pl.ds(g * L, L)]
                        fs = [w[0] * x0v + w[1] * x1v + w[2] * x2v + bd
                              for w, bd in zip(wrows, brows)]
                        for dr in range(DR):
                            blk[vt, dr, pl.ds(g * L, L)] = fs[dr]
                else:
                    tab = todv if dt < 8 else dowv
                    imap = tmap if dt < 8 else dmap
                    rows = STEPS_PER_DAY if dt < 8 else DOW
                    dbase = (dt - 4) * DR if dt < 8 else (dt - 8) * DR

                    @plsc.parallel_loop(0, VT * 8, unroll=2)
                    def gat_body(g2):
                        vt = g2 // 8
                        g = g2 % 8
                        iv = imap[pl.ds(g2 * L, L)]
                        rs = [plsc.load_gather(tab, [iv + (dbase + dr) * rows])
                              for dr in range(DR)]
                        for dr in range(DR):
                            blk[vt, dr, pl.ds(g * L, L)] = rs[dr]

                pltpu.async_copy(blk, out6.at[b, s, dt], sems[dt % 3])

            pltpu.make_async_copy(blks[0], out6.at[b, s, DT - 3], sems[0]).wait()
            pltpu.make_async_copy(blks[1], out6.at[b, s, DT - 2], sems[1]).wait()
            pltpu.make_async_copy(blks[2], out6.at[b, s, DT - 1], sems[2]).wait()
            return carry
        lax.fori_loop(0, BR, br_body, 0)
        return carry

    lax.fori_loop(0, SLABS_PER_W, slab_body, 0)


def kernel(x, W, b, tod_table, dow_table):
    x6 = (x.transpose(1, 3, 0, 2)
           .reshape(S, IN_DIM, BT, BR, VT, VC)
           .transpose(0, 1, 2, 4, 3, 5))
    wb = jnp.broadcast_to(W[:, :, None], (EMB, IN_DIM, L))
    bb = jnp.broadcast_to(b[:, None], (EMB, L))
    todf = tod_table.T.reshape(STEPS_PER_DAY * EMB)
    dowf = dow_table.T.reshape(DOW * EMB)

    mesh = plsc.VectorSubcoreMesh(core_axis_name="c", subcore_axis_name="s")
    run = pl.kernel(
        _tec_body,
        out_type=jax.ShapeDtypeStruct((B, S, DT, VT, DR, VC), jnp.float32),
        mesh=mesh,
        compiler_params=pltpu.CompilerParams(needs_layout_passes=False,
                                             use_tc_tiling_on_sc=False),
        scratch_types=[
            pltpu.VMEM((VT, BR, VC), jnp.float32),
            pltpu.VMEM((VT, BR, VC), jnp.float32),
            pltpu.VMEM((VT, BR, VC), jnp.float32),
            pltpu.VMEM((V,), jnp.int32),
            pltpu.VMEM((V,), jnp.int32),
            pltpu.VMEM((VT, DR, VC), jnp.float32),
            pltpu.VMEM((VT, DR, VC), jnp.float32),
            pltpu.VMEM((VT, DR, VC), jnp.float32),
            pltpu.VMEM((EMB, IN_DIM, L), jnp.float32),
            pltpu.VMEM((EMB, L), jnp.float32),
            pltpu.VMEM((STEPS_PER_DAY * EMB,), jnp.float32),
            pltpu.VMEM((DOW * EMB,), jnp.float32),
            pltpu.SemaphoreType.DMA,
            pltpu.SemaphoreType.DMA,
            pltpu.SemaphoreType.DMA,
            pltpu.SemaphoreType.DMA,
            pltpu.SemaphoreType.DMA,
        ],
    )
    out6 = run(x6, wb, bb, todf, dowf)
    return (out6.transpose(0, 1, 3, 5, 2, 4)
                .reshape(B, S, V, IN_DIM * EMB))


# speedup vs baseline: 1.2063x; 1.0075x over previous
import jax
import jax.numpy as jnp
from jax import lax
from jax.experimental import pallas as pl
from jax.experimental.pallas import tpu as pltpu
from jax.experimental.pallas import tpu_sc as plsc

IN_DIM = 3
EMB = 32
STEPS_PER_DAY = 288
DOW = 7
NC = 2
NS = 16
NW = NC * NS
L = 16

B, S, V = 64, 12, 1024
BT, BR = 8, 8
VT, VC = 8, 128
DT, DR = 12, 8
SLABS = S * BT
SLABS_PER_W = SLABS // NW


def _tec_body(x6, wb_hbm, bb_hbm, todf_hbm, dowf_hbm, out6,
              xb0, xb1, xb2, tmap, dmap, blk0, blk1, blk2,
              wv, bv, todv, dowv, semx, semt, sem0, sem1, sem2):
    wid = lax.axis_index("s") * NC + lax.axis_index("c")

    cp_tab = [
        pltpu.async_copy(wb_hbm, wv, semt),
        pltpu.async_copy(bb_hbm, bv, semt),
        pltpu.async_copy(todf_hbm, todv, semt),
        pltpu.async_copy(dowf_hbm, dowv, semt),
    ]
    for cp in cp_tab:
        cp.wait()

    iota = lax.iota(jnp.int32, L)
    blks = [blk0, blk1, blk2]
    sems = [sem0, sem1, sem2]

    def slab_body(k, carry):
        slab = wid * SLABS_PER_W + k
        s = slab // BT
        bt = slab % BT

        cpx = [
            pltpu.async_copy(x6.at[s, 0, bt], xb0, semx),
            pltpu.async_copy(x6.at[s, 1, bt], xb1, semx),
            pltpu.async_copy(x6.at[s, 2, bt], xb2, semx),
        ]
        for cp in cpx:
            cp.wait()

        def br_body(br, carry):
            b = bt * BR + br

            @plsc.parallel_loop(0, VT * 8, unroll=2)
            def idx_body(g2):
                vt = g2 // 8
                g = g2 % 8
                x1v = xb1[vt, br, pl.ds(g * L, L)]
                x2v = xb2[vt, br, pl.ds(g * L, L)]
                tmap[pl.ds(g2 * L, L)] = (x1v * float(STEPS_PER_DAY)).astype(jnp.int32)
                dmap[pl.ds(g2 * L, L)] = x2v.astype(jnp.int32)

            for dt in range(DT):
                blk = blks[dt % 3]
                if dt >= 3:
                    pltpu.make_async_copy(blk, out6.at[b, s, dt - 3],
                                          sems[dt % 3]).wait()
                if dt < 4:
                    wrows = [[wv[dt * DR + dr, j] for j in range(IN_DIM)]
                             for dr in range(DR)]
                    brows = [bv[dt * DR + dr] for dr in range(DR)]

                    @plsc.parallel_loop(0, VT * 8, unroll=2)
                    def feat_body(g2):
                        vt = g2 // 8
                        g = g2 % 8
                        x0v = xb0[vt, br, pl.ds(g * L, L)]
                        x1v = xb1[vt, br, pl.ds(g * L, L)]
                        x2v = xb2[vt, br, pl.ds(g * L, L)]
                        fs = [w[0] * x0v + w[1] * x1v + w[2] * x2v + bd
                              for w, bd in zip(wrows, brows)]
                        for dr in range(DR):
                            blk[vt, dr, pl.ds(g * L, L)] = fs[dr]
                else:
                    tab = todv if dt < 8 else dowv
                    imap = tmap if dt < 8 else dmap
                    rows = STEPS_PER_DAY if dt < 8 else 8
                    dbase = (dt - 4) * DR if dt < 8 else (dt - 8) * DR
                    tabs = [tab.at[pl.ds((dbase + dr) * rows, rows)]
                            for dr in range(DR)]

                    @plsc.parallel_loop(0, VT * 8, unroll=2)
                    def gat_body(g2):
                        vt = g2 // 8
                        g = g2 % 8
                        iv = imap[pl.ds(g2 * L, L)]
                        rs = [plsc.load_gather(tabs[dr], [iv])
                              for dr in range(DR)]
                        for dr in range(DR):
                            blk[vt, dr, pl.ds(g * L, L)] = rs[dr]

                pltpu.async_copy(blk, out6.at[b, s, dt], sems[dt % 3])

            pltpu.make_async_copy(blks[0], out6.at[b, s, DT - 3], sems[0]).wait()
            pltpu.make_async_copy(blks[1], out6.at[b, s, DT - 2], sems[1]).wait()
            pltpu.make_async_copy(blks[2], out6.at[b, s, DT - 1], sems[2]).wait()
            return carry
        lax.fori_loop(0, BR, br_body, 0)
        return carry

    lax.fori_loop(0, SLABS_PER_W, slab_body, 0)


def kernel(x, W, b, tod_table, dow_table):
    x6 = (x.transpose(1, 3, 0, 2)
           .reshape(S, IN_DIM, BT, BR, VT, VC)
           .transpose(0, 1, 2, 4, 3, 5))
    wb = jnp.broadcast_to(W[:, :, None], (EMB, IN_DIM, L))
    bb = jnp.broadcast_to(b[:, None], (EMB, L))
    todf = tod_table.T.reshape(STEPS_PER_DAY * EMB)
    dowf = jnp.pad(dow_table.T, ((0, 0), (0, 1))).reshape(8 * EMB)

    mesh = plsc.VectorSubcoreMesh(core_axis_name="c", subcore_axis_name="s")
    run = pl.kernel(
        _tec_body,
        out_type=jax.ShapeDtypeStruct((B, S, DT, VT, DR, VC), jnp.float32),
        mesh=mesh,
        compiler_params=pltpu.CompilerParams(needs_layout_passes=False,
                                             use_tc_tiling_on_sc=False),
        scratch_types=[
            pltpu.VMEM((VT, BR, VC), jnp.float32),
            pltpu.VMEM((VT, BR, VC), jnp.float32),
            pltpu.VMEM((VT, BR, VC), jnp.float32),
            pltpu.VMEM((V,), jnp.int32),
            pltpu.VMEM((V,), jnp.int32),
            pltpu.VMEM((VT, DR, VC), jnp.float32),
            pltpu.VMEM((VT, DR, VC), jnp.float32),
            pltpu.VMEM((VT, DR, VC), jnp.float32),
            pltpu.VMEM((EMB, IN_DIM, L), jnp.float32),
            pltpu.VMEM((EMB, L), jnp.float32),
            pltpu.VMEM((STEPS_PER_DAY * EMB,), jnp.float32),
            pltpu.VMEM((8 * EMB,), jnp.float32),
            pltpu.SemaphoreType.DMA,
            pltpu.SemaphoreType.DMA,
            pltpu.SemaphoreType.DMA,
            pltpu.SemaphoreType.DMA,
            pltpu.SemaphoreType.DMA,
        ],
    )
    out6 = run(x6, wb, bb, todf, dowf)
    return (out6.transpose(0, 1, 3, 5, 2, 4)
                .reshape(B, S, V, IN_DIM * EMB))


# speedup vs baseline: 1.2515x; 1.0375x over previous
import jax
import jax.numpy as jnp
from jax import lax
from jax.experimental import pallas as pl
from jax.experimental.pallas import tpu as pltpu
from jax.experimental.pallas import tpu_sc as plsc

IN_DIM = 3
EMB = 32
STEPS_PER_DAY = 288
DOW = 7
NC = 2
NS = 16
NW = NC * NS
L = 16

B, S, V = 64, 12, 1024
BT, BR = 8, 8
VT, VC = 8, 128
DT, DR = 12, 8
SLABS = S * BT
SLABS_PER_W = SLABS // NW


def _tec_body(x6, wb_hbm, bb_hbm, todf_hbm, dowf_hbm, out6,
              xb0, xb1, xb2, tmap, dmap, blk0, blk1, blk2,
              wv, bv, todv, dowv, semx, semt, sem0, sem1, sem2):
    wid = lax.axis_index("s") * NC + lax.axis_index("c")

    cp_tab = [
        pltpu.async_copy(wb_hbm, wv, semt),
        pltpu.async_copy(bb_hbm, bv, semt),
        pltpu.async_copy(todf_hbm, todv, semt),
        pltpu.async_copy(dowf_hbm, dowv, semt),
    ]
    for cp in cp_tab:
        cp.wait()

    iota = lax.iota(jnp.int32, L)
    blks = [blk0, blk1, blk2]
    sems = [sem0, sem1, sem2]

    def slab_body(k, carry):
        slab = wid * SLABS_PER_W + k
        s = slab // BT
        bt = slab % BT

        cpx = [
            pltpu.async_copy(x6.at[s, 0, bt], xb0, semx),
            pltpu.async_copy(x6.at[s, 1, bt], xb1, semx),
            pltpu.async_copy(x6.at[s, 2, bt], xb2, semx),
        ]
        for cp in cpx:
            cp.wait()

        def br_body(br, carry):
            b = bt * BR + br
            not_first = (k * BR + br) > 0

            @plsc.parallel_loop(0, VT * 8, unroll=2)
            def idx_body(g2):
                vt = g2 // 8
                g = g2 % 8
                x1v = xb1[vt, br, pl.ds(g * L, L)]
                x2v = xb2[vt, br, pl.ds(g * L, L)]
                tmap[pl.ds(g2 * L, L)] = (x1v * float(STEPS_PER_DAY)).astype(jnp.int32)
                dmap[pl.ds(g2 * L, L)] = x2v.astype(jnp.int32)

            for dt in range(DT):
                blk = blks[dt % 3]
                if dt >= 3:
                    pltpu.make_async_copy(blk, out6.at[b, s, dt - 3],
                                          sems[dt % 3]).wait()
                else:
                    @pl.when(not_first)
                    def _():
                        pltpu.make_async_copy(blk, out6.at[b, s, dt],
                                              sems[dt % 3]).wait()
                if dt < 4:
                    wrows = [[wv[dt * DR + dr, j] for j in range(IN_DIM)]
                             for dr in range(DR)]
                    brows = [bv[dt * DR + dr] for dr in range(DR)]

                    @plsc.parallel_loop(0, VT * 8, unroll=2)
                    def feat_body(g2):
                        vt = g2 // 8
                        g = g2 % 8
                        x0v = xb0[vt, br, pl.ds(g * L, L)]
                        x1v = xb1[vt, br, pl.ds(g * L, L)]
                        x2v = xb2[vt, br, pl.ds(g * L, L)]
                        fs = [w[0] * x0v + w[1] * x1v + w[2] * x2v + bd
                              for w, bd in zip(wrows, brows)]
                        for dr in range(DR):
                            blk[vt, dr, pl.ds(g * L, L)] = fs[dr]
                else:
                    tab = todv if dt < 8 else dowv
                    imap = tmap if dt < 8 else dmap
                    rows = STEPS_PER_DAY if dt < 8 else 8
                    dbase = (dt - 4) * DR if dt < 8 else (dt - 8) * DR
                    tabs = [tab.at[pl.ds((dbase + dr) * rows, rows)]
                            for dr in range(DR)]

                    @plsc.parallel_loop(0, VT * 8, unroll=2)
                    def gat_body(g2):
                        vt = g2 // 8
                        g = g2 % 8
                        iv = imap[pl.ds(g2 * L, L)]
                        rs = [plsc.load_gather(tabs[dr], [iv])
                              for dr in range(DR)]
                        for dr in range(DR):
                            blk[vt, dr, pl.ds(g * L, L)] = rs[dr]

                pltpu.async_copy(blk, out6.at[b, s, dt], sems[dt % 3])
            return carry
        lax.fori_loop(0, BR, br_body, 0)
        return carry

    lax.fori_loop(0, SLABS_PER_W, slab_body, 0)

    for j in range(3):
        pltpu.make_async_copy(blks[j], out6.at[0, 0, DT - 3 + j], sems[j]).wait()


def kernel(x, W, b, tod_table, dow_table):
    x6 = (x.transpose(1, 3, 0, 2)
           .reshape(S, IN_DIM, BT, BR, VT, VC)
           .transpose(0, 1, 2, 4, 3, 5))
    wb = jnp.broadcast_to(W[:, :, None], (EMB, IN_DIM, L))
    bb = jnp.broadcast_to(b[:, None], (EMB, L))
    todf = tod_table.T.reshape(STEPS_PER_DAY * EMB)
    dowf = jnp.pad(dow_table.T, ((0, 0), (0, 1))).reshape(8 * EMB)

    mesh = plsc.VectorSubcoreMesh(core_axis_name="c", subcore_axis_name="s")
    run = pl.kernel(
        _tec_body,
        out_type=jax.ShapeDtypeStruct((B, S, DT, VT, DR, VC), jnp.float32),
        mesh=mesh,
        compiler_params=pltpu.CompilerParams(needs_layout_passes=False,
                                             use_tc_tiling_on_sc=False),
        scratch_types=[
            pltpu.VMEM((VT, BR, VC), jnp.float32),
            pltpu.VMEM((VT, BR, VC), jnp.float32),
            pltpu.VMEM((VT, BR, VC), jnp.float32),
            pltpu.VMEM((V,), jnp.int32),
            pltpu.VMEM((V,), jnp.int32),
            pltpu.VMEM((VT, DR, VC), jnp.float32),
            pltpu.VMEM((VT, DR, VC), jnp.float32),
            pltpu.VMEM((VT, DR, VC), jnp.float32),
            pltpu.VMEM((EMB, IN_DIM, L), jnp.float32),
            pltpu.VMEM((EMB, L), jnp.float32),
            pltpu.VMEM((STEPS_PER_DAY * EMB,), jnp.float32),
            pltpu.VMEM((8 * EMB,), jnp.float32),
            pltpu.SemaphoreType.DMA,
            pltpu.SemaphoreType.DMA,
            pltpu.SemaphoreType.DMA,
            pltpu.SemaphoreType.DMA,
            pltpu.SemaphoreType.DMA,
        ],
    )
    out6 = run(x6, wb, bb, todf, dowf)
    return (out6.transpose(0, 1, 3, 5, 2, 4)
                .reshape(B, S, V, IN_DIM * EMB))


# speedup vs baseline: 1.2806x; 1.0232x over previous
import jax
import jax.numpy as jnp
from jax import lax
from jax.experimental import pallas as pl
from jax.experimental.pallas import tpu as pltpu
from jax.experimental.pallas import tpu_sc as plsc

IN_DIM = 3
EMB = 32
STEPS_PER_DAY = 288
DOW = 7
NC = 2
NS = 16
NW = NC * NS
L = 16

B, S, V = 64, 12, 1024
BT, BR = 8, 8
VT, VC = 8, 128
DT, DR = 12, 8
SLABS = S * BT
SLABS_PER_W = SLABS // NW


def _tec_body(x6, wb_hbm, bb_hbm, todf_hbm, dowf_hbm, out6,
              xb0, xb1, xb2, tmap, dmap, blk0, blk1, blk2,
              wv, bv, todv, dowv, semx, semt, sem0, sem1, sem2):
    wid = lax.axis_index("s") * NC + lax.axis_index("c")

    cp_tab = [
        pltpu.async_copy(wb_hbm, wv, semt),
        pltpu.async_copy(bb_hbm, bv, semt),
        pltpu.async_copy(todf_hbm, todv, semt),
        pltpu.async_copy(dowf_hbm, dowv, semt),
    ]
    for cp in cp_tab:
        cp.wait()

    iota = lax.iota(jnp.int32, L)
    blks = [blk0, blk1, blk2]
    sems = [sem0, sem1, sem2]

    def slab_body(k, carry):
        slab = wid * SLABS_PER_W + k
        s = slab // BT
        bt = slab % BT

        cpx = [
            pltpu.async_copy(x6.at[s, 0, bt], xb0, semx),
            pltpu.async_copy(x6.at[s, 1, bt], xb1, semx),
            pltpu.async_copy(x6.at[s, 2, bt], xb2, semx),
        ]
        for cp in cpx:
            cp.wait()

        def br_body(br, carry):
            b = bt * BR + br
            not_first = (k * BR + br) > 0

            @plsc.parallel_loop(0, VT * 8, unroll=2)
            def idx_body(g2):
                vt = g2 // 8
                g = g2 % 8
                x1v = xb1[vt, br, pl.ds(g * L, L)]
                x2v = xb2[vt, br, pl.ds(g * L, L)]
                tmap[pl.ds(g2 * L, L)] = (x1v * float(STEPS_PER_DAY)).astype(jnp.int32)
                dmap[pl.ds(g2 * L, L)] = x2v.astype(jnp.int32)

            for dt in range(DT):
                blk = blks[dt % 3]
                if dt >= 3:
                    pltpu.make_async_copy(blk, out6.at[b, s, dt - 3],
                                          sems[dt % 3]).wait()
                else:
                    @pl.when(not_first)
                    def _():
                        pltpu.make_async_copy(blk, out6.at[b, s, dt],
                                              sems[dt % 3]).wait()
                if dt < 4:
                    wrows = [[wv[dt * DR + dr, j] for j in range(IN_DIM)]
                             for dr in range(DR)]
                    brows = [bv[dt * DR + dr] for dr in range(DR)]

                    @plsc.parallel_loop(0, VT * 8, unroll=2)
                    def feat_body(g2):
                        vt = g2 // 8
                        g = g2 % 8
                        x0v = xb0[vt, br, pl.ds(g * L, L)]
                        x1v = xb1[vt, br, pl.ds(g * L, L)]
                        x2v = xb2[vt, br, pl.ds(g * L, L)]
                        fs = [w[0] * x0v + w[1] * x1v + w[2] * x2v + bd
                              for w, bd in zip(wrows, brows)]
                        for dr in range(DR):
                            blk[vt, dr, pl.ds(g * L, L)] = fs[dr]
                elif dt < 8:
                    dbase = (dt - 4) * DR
                    tabs = [todv.at[pl.ds((dbase + dr) * STEPS_PER_DAY,
                                          STEPS_PER_DAY)]
                            for dr in range(DR)]

                    @plsc.parallel_loop(0, VT * 8, unroll=2)
                    def gat_body(g2):
                        vt = g2 // 8
                        g = g2 % 8
                        iv = tmap[pl.ds(g2 * L, L)]
                        rs = [plsc.load_gather(tabs[dr], [iv])
                              for dr in range(DR)]
                        for dr in range(DR):
                            blk[vt, dr, pl.ds(g * L, L)] = rs[dr]
                else:
                    dbase = (dt - 8) * DR
                    rowregs = [dowv[pl.ds((dbase + dr) * 8, L)]
                               for dr in range(DR)]
                    dnums = lax.GatherDimensionNumbers(
                        offset_dims=(), collapsed_slice_dims=(0,),
                        start_index_map=(0,))

                    @plsc.parallel_loop(0, VT * 8, unroll=2)
                    def dow_body(g2):
                        vt = g2 // 8
                        g = g2 % 8
                        iv = dmap[pl.ds(g2 * L, L)]
                        rs = [lax.gather(
                                  rowregs[dr], iv[:, None], dnums, (1,),
                                  mode=lax.GatherScatterMode.PROMISE_IN_BOUNDS)
                              for dr in range(DR)]
                        for dr in range(DR):
                            blk[vt, dr, pl.ds(g * L, L)] = rs[dr]

                pltpu.async_copy(blk, out6.at[b, s, dt], sems[dt % 3])
            return carry
        lax.fori_loop(0, BR, br_body, 0)
        return carry

    lax.fori_loop(0, SLABS_PER_W, slab_body, 0)

    for j in range(3):
        pltpu.make_async_copy(blks[j], out6.at[0, 0, DT - 3 + j], sems[j]).wait()


def kernel(x, W, b, tod_table, dow_table):
    x6 = (x.transpose(1, 3, 0, 2)
           .reshape(S, IN_DIM, BT, BR, VT, VC)
           .transpose(0, 1, 2, 4, 3, 5))
    wb = jnp.broadcast_to(W[:, :, None], (EMB, IN_DIM, L))
    bb = jnp.broadcast_to(b[:, None], (EMB, L))
    todf = tod_table.T.reshape(STEPS_PER_DAY * EMB)
    dowf = jnp.pad(dow_table.T, ((0, 0), (0, 1))).reshape(8 * EMB)
    dowf = jnp.pad(dowf, (0, L))

    mesh = plsc.VectorSubcoreMesh(core_axis_name="c", subcore_axis_name="s")
    run = pl.kernel(
        _tec_body,
        out_type=jax.ShapeDtypeStruct((B, S, DT, VT, DR, VC), jnp.float32),
        mesh=mesh,
        compiler_params=pltpu.CompilerParams(needs_layout_passes=False,
                                             use_tc_tiling_on_sc=False),
        scratch_types=[
            pltpu.VMEM((VT, BR, VC), jnp.float32),
            pltpu.VMEM((VT, BR, VC), jnp.float32),
            pltpu.VMEM((VT, BR, VC), jnp.float32),
            pltpu.VMEM((V,), jnp.int32),
            pltpu.VMEM((V,), jnp.int32),
            pltpu.VMEM((VT, DR, VC), jnp.float32),
            pltpu.VMEM((VT, DR, VC), jnp.float32),
            pltpu.VMEM((VT, DR, VC), jnp.float32),
            pltpu.VMEM((EMB, IN_DIM, L), jnp.float32),
            pltpu.VMEM((EMB, L), jnp.float32),
            pltpu.VMEM((STEPS_PER_DAY * EMB,), jnp.float32),
            pltpu.VMEM((8 * EMB + L,), jnp.float32),
            pltpu.SemaphoreType.DMA,
            pltpu.SemaphoreType.DMA,
            pltpu.SemaphoreType.DMA,
            pltpu.SemaphoreType.DMA,
            pltpu.SemaphoreType.DMA,
        ],
    )
    out6 = run(x6, wb, bb, todf, dowf)
    return (out6.transpose(0, 1, 3, 5, 2, 4)
                .reshape(B, S, V, IN_DIM * EMB))


# speedup vs baseline: 1.2812x; 1.0005x over previous
import jax
import jax.numpy as jnp
from jax import lax
from jax.experimental import pallas as pl
from jax.experimental.pallas import tpu as pltpu
from jax.experimental.pallas import tpu_sc as plsc

IN_DIM = 3
EMB = 32
STEPS_PER_DAY = 288
NC = 2
NS = 16
NW = NC * NS
L = 16

B, S, V = 64, 12, 1024
BT, BR = 8, 8
VT, VC = 8, 128
DT, DR = 12, 8
SLABS = S * BT
SLABS_PER_W = SLABS // NW


def _tec_body(x6, wb_hbm, bb_hbm, todf_hbm, dowf_hbm, out6,
              xb0, xb1, xb2, tmap, dmap, blk0, blk1, blk2,
              wv, bv, todv, dowv, semx, semt, sem0, sem1, sem2):
    wid = lax.axis_index("s") * NC + lax.axis_index("c")

    cp_tab = [
        pltpu.async_copy(wb_hbm, wv, semt),
        pltpu.async_copy(bb_hbm, bv, semt),
        pltpu.async_copy(todf_hbm, todv, semt),
        pltpu.async_copy(dowf_hbm, dowv, semt),
    ]
    for cp in cp_tab:
        cp.wait()

    blks = [blk0, blk1, blk2]
    sems = [sem0, sem1, sem2]

    def slab_body(k, carry):
        slab = wid * SLABS_PER_W + k
        s = slab // BT
        bt = slab % BT

        cpx = [
            pltpu.async_copy(x6.at[s, 0, bt], xb0, semx),
            pltpu.async_copy(x6.at[s, 1, bt], xb1, semx),
            pltpu.async_copy(x6.at[s, 2, bt], xb2, semx),
        ]
        for cp in cpx:
            cp.wait()

        def br_body(br, carry):
            b = bt * BR + br
            not_first = (k * BR + br) > 0

            @plsc.parallel_loop(0, VT * 8, unroll=2)
            def idx_body(g2):
                vt = g2 // 8
                g = g2 % 8
                x1v = xb1[vt, br, pl.ds(g * L, L)]
                x2v = xb2[vt, br, pl.ds(g * L, L)]
                tmap[pl.ds(g2 * L, L)] = (x1v * float(STEPS_PER_DAY)).astype(jnp.int32)
                dmap[pl.ds(g2 * L, L)] = x2v.astype(jnp.int32)

            for dt in range(DT):
                blk = blks[dt % 3]
                if dt >= 3:
                    pltpu.make_async_copy(blk, out6.at[b, s, dt - 3],
                                          sems[dt % 3]).wait()
                else:
                    @pl.when(not_first)
                    def _():
                        pltpu.make_async_copy(blk, out6.at[b, s, dt],
                                              sems[dt % 3]).wait()
                if dt < 4:
                    wrows = [[wv[dt * DR + dr, j] for j in range(IN_DIM)]
                             for dr in range(DR)]
                    brows = [bv[dt * DR + dr] for dr in range(DR)]

                    @plsc.parallel_loop(0, VT * 8, unroll=2)
                    def feat_body(g2):
                        vt = g2 // 8
                        g = g2 % 8
                        x0v = xb0[vt, br, pl.ds(g * L, L)]
                        x1v = xb1[vt, br, pl.ds(g * L, L)]
                        x2v = xb2[vt, br, pl.ds(g * L, L)]
                        fs = [w[0] * x0v + w[1] * x1v + w[2] * x2v + bd
                              for w, bd in zip(wrows, brows)]
                        for dr in range(DR):
                            blk[vt, dr, pl.ds(g * L, L)] = fs[dr]
                elif dt < 8:
                    dbase = (dt - 4) * DR
                    tabs = [todv.at[pl.ds((dbase + dr) * STEPS_PER_DAY,
                                          STEPS_PER_DAY)]
                            for dr in range(DR)]

                    @plsc.parallel_loop(0, VT * 8, unroll=2)
                    def gat_body(g2):
                        vt = g2 // 8
                        g = g2 % 8
                        iv = tmap[pl.ds(g2 * L, L)]
                        rs = [plsc.load_gather(tabs[dr], [iv])
                              for dr in range(DR)]
                        for dr in range(DR):
                            blk[vt, dr, pl.ds(g * L, L)] = rs[dr]
                else:
                    dbase = (dt - 8) * DR
                    rowregs = [dowv[pl.ds((dbase + dr) * 8, L)]
                               for dr in range(DR)]
                    dnums = lax.GatherDimensionNumbers(
                        offset_dims=(), collapsed_slice_dims=(0,),
                        start_index_map=(0,))

                    @plsc.parallel_loop(0, VT * 8, unroll=2)
                    def dow_body(g2):
                        vt = g2 // 8
                        g = g2 % 8
                        iv = dmap[pl.ds(g2 * L, L)]
                        rs = [lax.gather(
                                  rowregs[dr], iv[:, None], dnums, (1,),
                                  mode=lax.GatherScatterMode.PROMISE_IN_BOUNDS)
                              for dr in range(DR)]
                        for dr in range(DR):
                            blk[vt, dr, pl.ds(g * L, L)] = rs[dr]

                pltpu.async_copy(blk, out6.at[b, s, dt], sems[dt % 3])
            return carry
        lax.fori_loop(0, BR, br_body, 0)
        return carry

    lax.fori_loop(0, SLABS_PER_W, slab_body, 0)

    for j in range(3):
        pltpu.make_async_copy(blks[j], out6.at[0, 0, DT - 3 + j], sems[j]).wait()


def kernel(x, W, b, tod_table, dow_table):
    x6 = (x.transpose(1, 3, 0, 2)
           .reshape(S, IN_DIM, BT, BR, VT, VC)
           .transpose(0, 1, 2, 4, 3, 5))
    wb = jnp.broadcast_to(W[:, :, None], (EMB, IN_DIM, L))
    bb = jnp.broadcast_to(b[:, None], (EMB, L))
    todf = tod_table.T.reshape(STEPS_PER_DAY * EMB)
    dowf = jnp.pad(dow_table.T, ((0, 0), (0, 1))).reshape(8 * EMB)
    dowf = jnp.pad(dowf, (0, L))

    mesh = plsc.VectorSubcoreMesh(core_axis_name="c", subcore_axis_name="s")
    run = pl.kernel(
        _tec_body,
        out_type=jax.ShapeDtypeStruct((B, S, DT, VT, DR, VC), jnp.float32),
        mesh=mesh,
        compiler_params=pltpu.CompilerParams(needs_layout_passes=False,
                                             use_tc_tiling_on_sc=False),
        scratch_types=[
            pltpu.VMEM((VT, BR, VC), jnp.float32),
            pltpu.VMEM((VT, BR, VC), jnp.float32),
            pltpu.VMEM((VT, BR, VC), jnp.float32),
            pltpu.VMEM((V,), jnp.int32),
            pltpu.VMEM((V,), jnp.int32),
            pltpu.VMEM((VT, DR, VC), jnp.float32),
            pltpu.VMEM((VT, DR, VC), jnp.float32),
            pltpu.VMEM((VT, DR, VC), jnp.float32),
            pltpu.VMEM((EMB, IN_DIM, L), jnp.float32),
            pltpu.VMEM((EMB, L), jnp.float32),
            pltpu.VMEM((STEPS_PER_DAY * EMB,), jnp.float32),
            pltpu.VMEM((8 * EMB + L,), jnp.float32),
            pltpu.SemaphoreType.DMA,
            pltpu.SemaphoreType.DMA,
            pltpu.SemaphoreType.DMA,
            pltpu.SemaphoreType.DMA,
            pltpu.SemaphoreType.DMA,
        ],
    )
    out6 = run(x6, wb, bb, todf, dowf)
    return (out6.transpose(0, 1, 3, 5, 2, 4)
                .reshape(B, S, V, IN_DIM * EMB))
